# baseline (speedup 1.0000x reference)
import jax
import jax.numpy as jnp
from jax.experimental import pallas as pl

N = 10000
E = 160000
H = 8
F = 64
L = 2


def _lin(x, w, b=None):
    y = x @ w.T
    if b is not None:
        y = y + b
    return y


def _lrelu(x):
    return jnp.where(x >= 0, x, 0.2 * x)


def _enc_body(x_ref, w_ref, b_ref, o_ref):
    o_ref[...] = jax.nn.relu(
        jnp.dot(x_ref[...], w_ref[...], preferred_element_type=jnp.float32)
        + b_ref[...]
    )


def _node_enc(x, w, b):
    NP = 10240
    OP = 256
    xp = jnp.pad(x, ((0, NP - N), (0, 0)))
    wt = jnp.pad(w.T, ((0, 0), (0, OP - w.shape[0])))
    bp = jnp.pad(b, (0, OP - b.shape[0]))
    out = pl.pallas_call(
        _enc_body,
        out_shape=jax.ShapeDtypeStruct((NP, OP), jnp.float32),
        grid=(NP // 1024,),
        in_specs=[
            pl.BlockSpec((1024, 256), lambda i: (i, 0)),
            pl.BlockSpec((256, OP), lambda i: (0, 0)),
            pl.BlockSpec((OP,), lambda i: (0,)),
        ],
        out_specs=pl.BlockSpec((1024, OP), lambda i: (i, 0)),
    )(xp, wt, bp)
    return out[:N, : w.shape[0]]


def kernel(x, edge_index, edge_attr, params):
    src = edge_index[0]
    dst = edge_index[1]
    h = _node_enc(x, params['node_enc_w'], params['node_enc_b'])
    h_last = None
    for i in range(L):
        ef = jax.nn.relu(_lin(edge_attr, params[f'edge_enc_w_{i}'], params[f'edge_enc_b_{i}']))
        fsrc = _lin(h, params[f'src_fc_w_{i}']).reshape(N, H, F)
        asrc = _lin(h, params[f'attn_src_w_{i}']).reshape(N, H, 1)
        adst = _lin(h, params[f'attn_dst_w_{i}']).reshape(N, H, 1)
        e = asrc[src] + adst[dst] + _lin(ef, params[f'attn_edge_w_{i}']).reshape(E, H, 1)
        e = _lrelu(e)
        m = fsrc[src] * e
        agg = jax.ops.segment_sum(m, dst, num_segments=N)
        mean = agg.mean(-1, keepdims=True)
        var = agg.var(-1, keepdims=True) + 1e-9
        hn = (agg - mean) * params[f'scale_{i}'] * jax.lax.rsqrt(var) + params[f'offset_{i}']
        hn = _lin(hn.reshape(N, H * F), params[f'agg_w_{i}'], params[f'agg_b_{i}']).reshape(N, H, F)
        rst = hn + _lin(h, params[f'dst_w_{i}'], params[f'dst_b_{i}']).reshape(N, H, F)
        hf = rst.reshape(N, H * F)
        if h_last is not None:
            hf = hf + h_last
        h_last = hf
        mu = hf.mean(0)
        v = hf.var(0)
        hf = (hf - mu) / jnp.sqrt(v + 1e-5) * params[f'bn_g_{i}'] + params[f'bn_b_{i}']
        h = jax.nn.relu(hf)
    return _lin(h, params['pred_w'], params['pred_b'])


# speedup vs baseline: 11.0467x; 11.0467x over previous
import functools

import jax
import jax.numpy as jnp
from jax import lax
from jax.experimental import pallas as pl
from jax.experimental.pallas import tpu as pltpu
from jax.experimental.pallas import tpu_sc as plsc

N = 10000
NP = 10240
NRANGES = 64
RN = 160
E = 160000
EP = 160768
C = 32
H = 8
F = 64
HF = 512
D_IN = 256
FIRST = 150
K0 = 256
NB = 1024
G = NP // NB
NC_OUT = 40
FEXT = 640

_f32 = jnp.float32
_i32 = jnp.int32


def _enc_body(x_ref, w_ref, b_ref, o_ref):
    o_ref[...] = jax.nn.relu(
        jnp.dot(x_ref[...], w_ref[...], preferred_element_type=_f32) + b_ref[...]
    )


def _node_enc(xp, wt, bp):
    return pl.pallas_call(
        _enc_body,
        out_shape=jax.ShapeDtypeStruct((NP, K0), _f32),
        grid=(G,),
        in_specs=[
            pl.BlockSpec((NB, D_IN), lambda i: (i, 0)),
            pl.BlockSpec((D_IN, K0), lambda i: (0, 0)),
            pl.BlockSpec((K0,), lambda i: (0,)),
        ],
        out_specs=pl.BlockSpec((NB, K0), lambda i: (i, 0)),
    )(xp, wt, bp)


def _proj_body(h_ref, wfe_ref, wad_ref, f_ref, ad_ref):
    hv = h_ref[...]
    f_ref[...] = jnp.dot(hv, wfe_ref[...], preferred_element_type=_f32)
    ad_ref[...] = jnp.dot(hv, wad_ref[...], preferred_element_type=_f32)


def _proj(h, wfe, wad):
    k = h.shape[1]
    return pl.pallas_call(
        _proj_body,
        out_shape=[
            jax.ShapeDtypeStruct((NP, FEXT), _f32),
            jax.ShapeDtypeStruct((NP, H), _f32),
        ],
        grid=(G,),
        in_specs=[
            pl.BlockSpec((NB, k), lambda i: (i, 0)),
            pl.BlockSpec((k, FEXT), lambda i: (0, 0)),
            pl.BlockSpec((k, H), lambda i: (0, 0)),
        ],
        out_specs=[
            pl.BlockSpec((NB, FEXT), lambda i: (i, 0)),
            pl.BlockSpec((NB, H), lambda i: (i, 0)),
        ],
    )(h, wfe, wad)


def _eatt_body(ea_ref, w1_ref, b1_ref, w2_ref, o_ref):
    t = jax.nn.relu(
        jnp.dot(ea_ref[...], w1_ref[...], preferred_element_type=_f32) + b1_ref[...]
    )
    o_ref[...] = jnp.dot(t, w2_ref[...], preferred_element_type=_f32)


def _eatt(ea_p, w1t, b1, w2t):
    return pl.pallas_call(
        _eatt_body,
        out_shape=jax.ShapeDtypeStruct((EP, H), _f32),
        grid=(EP // NB,),
        in_specs=[
            pl.BlockSpec((NB, 16), lambda i: (i, 0)),
            pl.BlockSpec((16, 16), lambda i: (0, 0)),
            pl.BlockSpec((16,), lambda i: (0,)),
            pl.BlockSpec((16, H), lambda i: (0, 0)),
        ],
        out_specs=pl.BlockSpec((NB, H), lambda i: (i, 0)),
    )(ea_p, w1t, b1, w2t)


def _c1_body(has_last, agg_ref, h_ref, hl_ref, one8_ref, exp8_ref, sc_ref, of_ref,
             wagg_ref, bagg_ref, wdst_ref, bdst_ref, hf_ref, ps_ref, pq_ref):
    a = agg_ref[...]
    m8 = jnp.dot(a, one8_ref[...], preferred_element_type=_f32) * (1.0 / F)
    q8 = jnp.dot(a * a, one8_ref[...], preferred_element_type=_f32) * (1.0 / F)
    mean = jnp.dot(m8, exp8_ref[...], preferred_element_type=_f32)
    msq = jnp.dot(q8, exp8_ref[...], preferred_element_type=_f32)
    var = msq - mean * mean + 1e-9
    hn = (a - mean) * sc_ref[...] * lax.rsqrt(var) + of_ref[...]
    o = (jnp.dot(hn, wagg_ref[...], preferred_element_type=_f32) + bagg_ref[...]
         + jnp.dot(h_ref[...], wdst_ref[...], preferred_element_type=_f32)
         + bdst_ref[...])
    if has_last:
        o = o + hl_ref[...]
    hf_ref[...] = o
    gi = pl.program_id(0)
    rows = lax.broadcasted_iota(_i32, (NB, 1), 0) + gi * NB
    om = jnp.where(rows < N, o, 0.0)
    ps_ref[...] = jnp.sum(om, axis=0, keepdims=True).reshape(1, 1, HF)
    pq_ref[...] = jnp.sum(om * om, axis=0, keepdims=True).reshape(1, 1, HF)


def _c1(agg, h, hl, one8, exp8, scale, offset, wagg, bagg, wdst, bdst):
    k = h.shape[1]
    has_last = hl is not None
    args = [agg, h] + ([hl] if has_last else []) + [
        one8, exp8, scale, offset, wagg, bagg, wdst, bdst]
    in_specs = [
        pl.BlockSpec((NB, HF), lambda i: (i, 0)),
        pl.BlockSpec((NB, k), lambda i: (i, 0)),
    ] + ([pl.BlockSpec((NB, HF), lambda i: (i, 0))] if has_last else []) + [
        pl.BlockSpec((HF, H), lambda i: (0, 0)),
        pl.BlockSpec((H, HF), lambda i: (0, 0)),
        pl.BlockSpec((1, HF), lambda i: (0, 0)),
        pl.BlockSpec((1, HF), lambda i: (0, 0)),
        pl.BlockSpec((HF, HF), lambda i: (0, 0)),
        pl.BlockSpec((HF,), lambda i: (0,)),
        pl.BlockSpec((k, HF), lambda i: (0, 0)),
        pl.BlockSpec((HF,), lambda i: (0,)),
    ]
    body = functools.partial(_c1_body, has_last) if has_last else _c1_nolast
    return pl.pallas_call(
        body,
        out_shape=[
            jax.ShapeDtypeStruct((NP, HF), _f32),
            jax.ShapeDtypeStruct((G, 1, HF), _f32),
            jax.ShapeDtypeStruct((G, 1, HF), _f32),
        ],
        grid=(G,),
        in_specs=in_specs,
        out_specs=[
            pl.BlockSpec((NB, HF), lambda i: (i, 0)),
            pl.BlockSpec((1, 1, HF), lambda i: (i, 0, 0)),
            pl.BlockSpec((1, 1, HF), lambda i: (i, 0, 0)),
        ],
    )(*args)


def _c1_nolast(agg_ref, h_ref, one8_ref, exp8_ref, sc_ref, of_ref,
               wagg_ref, bagg_ref, wdst_ref, bdst_ref, hf_ref, ps_ref, pq_ref):
    _c1_body(False, agg_ref, h_ref, None, one8_ref, exp8_ref, sc_ref, of_ref,
             wagg_ref, bagg_ref, wdst_ref, bdst_ref, hf_ref, ps_ref, pq_ref)


def _c2_body(hf_ref, ps_ref, pq_ref, g_ref, b_ref, o_ref):
    mu = jnp.sum(ps_ref[...][:, 0, :], axis=0) * (1.0 / N)
    ex2 = jnp.sum(pq_ref[...][:, 0, :], axis=0) * (1.0 / N)
    var = ex2 - mu * mu
    s = g_ref[...][0] / jnp.sqrt(var + 1e-5)
    o_ref[...] = jnp.maximum((hf_ref[...] - mu) * s + b_ref[...][0], 0.0)


def _c2(hf, ps, pq, g, b):
    return pl.pallas_call(
        _c2_body,
        out_shape=jax.ShapeDtypeStruct((NP, HF), _f32),
        grid=(G,),
        in_specs=[
            pl.BlockSpec((NB, HF), lambda i: (i, 0)),
            pl.BlockSpec((G, 1, HF), lambda i: (0, 0, 0)),
            pl.BlockSpec((G, 1, HF), lambda i: (0, 0, 0)),
            pl.BlockSpec((1, HF), lambda i: (0, 0)),
            pl.BlockSpec((1, HF), lambda i: (0, 0)),
        ],
        out_specs=pl.BlockSpec((NB, HF), lambda i: (i, 0)),
    )(hf, ps, pq, g, b)


def _pred_body(h_ref, w_ref, b_ref, o_ref):
    o_ref[...] = jnp.dot(h_ref[...], w_ref[...], preferred_element_type=_f32) + b_ref[...]


def _pred(h, wt, bp):
    return pl.pallas_call(
        _pred_body,
        out_shape=jax.ShapeDtypeStruct((NP, 128), _f32),
        grid=(G,),
        in_specs=[
            pl.BlockSpec((NB, HF), lambda i: (i, 0)),
            pl.BlockSpec((HF, 128), lambda i: (0, 0)),
            pl.BlockSpec((128,), lambda i: (0,)),
        ],
        out_specs=pl.BlockSpec((NB, 128), lambda i: (i, 0)),
    )(h, wt, bp)


def _sc_agg_body(fsrc_hbm, adst_hbm, eatt_hbm, src_hbm, dstloc_hbm,
                 starts_hbm, out_hbm,
                 acc, rows, eatt_b, adst_rng, srcidx, dstloc_b,
                 coeff_b, starts_v, sem_rows):
    ncores = 2
    wid = lax.axis_index("s") * ncores + lax.axis_index("c")
    pltpu.sync_copy(starts_hbm, starts_v)
    zero16 = jnp.zeros((16,), _f32)
    iota = lax.iota(_i32, 16)
    low = iota < 8
    col8 = lax.bitwise_and(iota, 7)

    for p in range(2):
        r = wid * 2 + p

        def zrow(i, _):
            for vv in range(HF // 16):
                acc[i, pl.ds(vv * 16, 16)] = zero16
            return 0
        lax.fori_loop(0, RN, zrow, 0, unroll=False)

        pltpu.sync_copy(adst_hbm.at[pl.ds(pl.multiple_of(r * RN, 8), RN)],
                        adst_rng)
        sv = starts_v[pl.ds(r, 16)]
        es = sv[0]
        ee = sv[1]
        base0 = es - lax.rem(es, 8)
        nchunks = lax.div(ee - base0 + (C - 1), C)

        def chunk_body(kk, _):
            cbase = pl.multiple_of(base0 + kk * C, 8)
            pltpu.sync_copy(src_hbm.at[pl.ds(cbase, C)], srcidx)
            pltpu.sync_copy(dstloc_hbm.at[pl.ds(cbase, C)],
                            dstloc_b.at[pl.ds(0, C)])
            pltpu.sync_copy(eatt_hbm.at[pl.ds(cbase, C)], eatt_b)
            cp_rows = pltpu.async_copy(fsrc_hbm.at[srcidx], rows, sem_rows)
            jlo = jnp.maximum(es - cbase, 0)
            jhi = jnp.minimum(ee - cbase, C)
            cp_rows.wait()

            def pair_body(pp, _):
                rlo = jnp.where(low, 2 * pp, 2 * pp + 1)
                a = plsc.load_gather(rows, [rlo, col8 + HF])
                ev = plsc.load_gather(eatt_b, [rlo, col8])
                dv16 = dstloc_b[pl.ds(2 * pp, 16)]
                drow = jnp.where(low, dv16[0], dv16[1])
                dv = plsc.load_gather(adst_rng, [drow, col8])
                s = a + dv + ev
                coeff_b[pl.ds(pp * 16, 16)] = jnp.maximum(s, 0.2 * s)
                return 0

            lax.fori_loop(lax.div(jlo, 2), lax.div(jhi + 1, 2), pair_body, 0,
                          unroll=False)

            def edge_body(j, _):
                dl = dstloc_b[pl.ds(j, 16)][0]
                cvec = coeff_b[pl.ds(j * 8, 16)]
                for hh in range(H):
                    cv = lax.broadcast(cvec[hh], (16,))
                    for vv in range(4):
                        off = hh * 64 + vv * 16
                        plsc.addupdate(acc.at[dl, pl.ds(off, 16)],
                                       cv * rows[j, pl.ds(off, 16)])
                return 0

            lax.fori_loop(jlo, jhi, edge_body, 0, unroll=False)
            return 0

        lax.fori_loop(0, nchunks, chunk_body, 0, unroll=False)
        pltpu.sync_copy(acc, out_hbm.at[r])


def _sc_agg(fsrc, adst, eatt, src_p, dstloc_p, starts_p):
    return pl.kernel(
        _sc_agg_body,
        out_type=jax.ShapeDtypeStruct((NRANGES, RN, HF), _f32),
        mesh=plsc.VectorSubcoreMesh(core_axis_name="c", subcore_axis_name="s",
                                    num_cores=2, num_subcores=16),
        compiler_params=pltpu.CompilerParams(needs_layout_passes=False),
        scratch_types=[
            pltpu.VMEM((RN, HF), _f32),
            pltpu.VMEM((C, FEXT), _f32),
            pltpu.VMEM((C, H), _f32),
            pltpu.VMEM((RN, H), _f32),
            pltpu.VMEM((C,), _i32),
            pltpu.VMEM((C + 16,), _i32),
            pltpu.VMEM((C * H + 16,), _f32),
            pltpu.VMEM((80,), _i32),
            pltpu.SemaphoreType.DMA,
        ],
    )(fsrc, adst, eatt, src_p, dstloc_p, starts_p)


def _padw(w, k):
    wt = w.T
    if wt.shape[0] < k:
        wt = jnp.pad(wt, ((0, k - wt.shape[0]), (0, 0)))
    return wt


def kernel(x, edge_index, edge_attr, params):
    src = edge_index[0]
    dst = edge_index[1]

    order = jnp.argsort(dst)
    dst_s = dst[order]
    src_p = jnp.pad(src[order], (0, EP - E))
    dstloc_p = jnp.pad(dst_s % RN, (0, EP - E))
    ea_p = jnp.pad(edge_attr[order], ((0, EP - E), (0, 0)))
    starts = jnp.searchsorted(dst_s, jnp.arange(65, dtype=_i32) * RN).astype(_i32)
    starts_p = jnp.pad(starts, (0, 15))

    xp = jnp.pad(x, ((0, NP - N), (0, 0)))
    p = params

    one8 = (jnp.arange(HF, dtype=_i32)[:, None] // F
            == jnp.arange(H, dtype=_i32)[None, :]).astype(_f32)
    exp8 = one8.T

    enc_wt = jnp.pad(p['node_enc_w'].T, ((0, 0), (0, K0 - FIRST)))
    enc_b = jnp.pad(p['node_enc_b'], (0, K0 - FIRST))
    h = _node_enc(xp, enc_wt, enc_b)

    h_last = None
    for i in range(2):
        k = K0 if i == 0 else HF
        wf = _padw(p[f'src_fc_w_{i}'], k)
        was = _padw(p[f'attn_src_w_{i}'], k)
        wad = _padw(p[f'attn_dst_w_{i}'], k)
        wfe = jnp.concatenate([wf, was, jnp.zeros((k, FEXT - HF - H), _f32)],
                              axis=1)
        fsrc, adst = _proj(h, wfe, wad)

        w1t = p[f'edge_enc_w_{i}'].T
        b1 = p[f'edge_enc_b_{i}']
        w2t = p[f'attn_edge_w_{i}'].T
        eatt = _eatt(ea_p, w1t, b1, w2t)

        agg = _sc_agg(fsrc, adst, eatt, src_p, dstloc_p,
                      starts_p).reshape(NP, HF)

        scale = p[f'scale_{i}'].reshape(1, HF)
        offset = p[f'offset_{i}'].reshape(1, HF)
        wagg = p[f'agg_w_{i}'].T
        bagg = p[f'agg_b_{i}']
        wdst = _padw(p[f'dst_w_{i}'], k)
        bdst = p[f'dst_b_{i}']
        hf, ps, pq = _c1(agg, h, h_last, one8, exp8, scale, offset,
                         wagg, bagg, wdst, bdst)
        h = _c2(hf, ps, pq, p[f'bn_g_{i}'].reshape(1, HF),
                p[f'bn_b_{i}'].reshape(1, HF))
        h_last = hf

    pred_wt = jnp.pad(p['pred_w'].T, ((0, 0), (0, 128 - NC_OUT)))
    pred_b = jnp.pad(p['pred_b'], (0, 128 - NC_OUT))
    out = _pred(h, pred_wt, pred_b)
    return out[:N, :NC_OUT]


# speedup vs baseline: 14.6268x; 1.3241x over previous
import functools

import jax
import jax.numpy as jnp
from jax import lax
from jax.experimental import pallas as pl
from jax.experimental.pallas import tpu as pltpu
from jax.experimental.pallas import tpu_sc as plsc

N = 10000
NP = 10240
NRANGES = 64
RN = 160
E = 160000
EP = 160768
C = 16
H = 8
F = 64
HF = 512
D_IN = 256
FIRST = 150
K0 = 256
NB = 1024
G = NP // NB
NC_OUT = 40
FEXT = 640

_f32 = jnp.float32
_i32 = jnp.int32


def _enc_body(x_ref, w_ref, b_ref, o_ref):
    o_ref[...] = jax.nn.relu(
        jnp.dot(x_ref[...], w_ref[...], preferred_element_type=_f32) + b_ref[...]
    )


def _node_enc(xp, wt, bp):
    return pl.pallas_call(
        _enc_body,
        out_shape=jax.ShapeDtypeStruct((NP, K0), _f32),
        grid=(G,),
        in_specs=[
            pl.BlockSpec((NB, D_IN), lambda i: (i, 0)),
            pl.BlockSpec((D_IN, K0), lambda i: (0, 0)),
            pl.BlockSpec((K0,), lambda i: (0,)),
        ],
        out_specs=pl.BlockSpec((NB, K0), lambda i: (i, 0)),
    )(xp, wt, bp)


def _proj_body(h_ref, wfe_ref, wad_ref, f_ref, ad_ref):
    hv = h_ref[...]
    f_ref[...] = jnp.dot(hv, wfe_ref[...], preferred_element_type=_f32)
    ad_ref[...] = jnp.dot(hv, wad_ref[...], preferred_element_type=_f32)


def _proj(h, wfe, wad):
    k = h.shape[1]
    return pl.pallas_call(
        _proj_body,
        out_shape=[
            jax.ShapeDtypeStruct((NP, FEXT), _f32),
            jax.ShapeDtypeStruct((NP, H), _f32),
        ],
        grid=(G,),
        in_specs=[
            pl.BlockSpec((NB, k), lambda i: (i, 0)),
            pl.BlockSpec((k, FEXT), lambda i: (0, 0)),
            pl.BlockSpec((k, H), lambda i: (0, 0)),
        ],
        out_specs=[
            pl.BlockSpec((NB, FEXT), lambda i: (i, 0)),
            pl.BlockSpec((NB, H), lambda i: (i, 0)),
        ],
    )(h, wfe, wad)


def _eatt_body(ea_ref, dl_ref, w1_ref, b1_ref, w2_ref, o_ref):
    t = jax.nn.relu(
        jnp.dot(ea_ref[...], w1_ref[...], preferred_element_type=_f32) + b1_ref[...]
    )
    ea = jnp.dot(t, w2_ref[...], preferred_element_type=_f32)
    o_ref[...] = jnp.concatenate(
        [dl_ref[...], ea, jnp.zeros((NB, 7), _f32)], axis=1)


def _eatt(ea_p, dl_f, w1t, b1, w2t):
    return pl.pallas_call(
        _eatt_body,
        out_shape=jax.ShapeDtypeStruct((EP, 16), _f32),
        grid=(EP // NB,),
        in_specs=[
            pl.BlockSpec((NB, 16), lambda i: (i, 0)),
            pl.BlockSpec((NB, 1), lambda i: (i, 0)),
            pl.BlockSpec((16, 16), lambda i: (0, 0)),
            pl.BlockSpec((16,), lambda i: (0,)),
            pl.BlockSpec((16, H), lambda i: (0, 0)),
        ],
        out_specs=pl.BlockSpec((NB, 16), lambda i: (i, 0)),
    )(ea_p, dl_f, w1t, b1, w2t)


def _c1_body(has_last, agg_ref, h_ref, hl_ref, one8_ref, exp8_ref, sc_ref, of_ref,
             wagg_ref, bagg_ref, wdst_ref, bdst_ref, hf_ref, ps_ref, pq_ref):
    a = agg_ref[...]
    m8 = jnp.dot(a, one8_ref[...], preferred_element_type=_f32) * (1.0 / F)
    q8 = jnp.dot(a * a, one8_ref[...], preferred_element_type=_f32) * (1.0 / F)
    mean = jnp.dot(m8, exp8_ref[...], preferred_element_type=_f32)
    msq = jnp.dot(q8, exp8_ref[...], preferred_element_type=_f32)
    var = msq - mean * mean + 1e-9
    hn = (a - mean) * sc_ref[...] * lax.rsqrt(var) + of_ref[...]
    o = (jnp.dot(hn, wagg_ref[...], preferred_element_type=_f32) + bagg_ref[...]
         + jnp.dot(h_ref[...], wdst_ref[...], preferred_element_type=_f32)
         + bdst_ref[...])
    if has_last:
        o = o + hl_ref[...]
    hf_ref[...] = o
    gi = pl.program_id(0)
    rows = lax.broadcasted_iota(_i32, (NB, 1), 0) + gi * NB
    om = jnp.where(rows < N, o, 0.0)
    ps_ref[...] = jnp.sum(om, axis=0, keepdims=True).reshape(1, 1, HF)
    pq_ref[...] = jnp.sum(om * om, axis=0, keepdims=True).reshape(1, 1, HF)


def _c1(agg, h, hl, one8, exp8, scale, offset, wagg, bagg, wdst, bdst):
    k = h.shape[1]
    has_last = hl is not None
    args = [agg, h] + ([hl] if has_last else []) + [
        one8, exp8, scale, offset, wagg, bagg, wdst, bdst]
    in_specs = [
        pl.BlockSpec((NB, HF), lambda i: (i, 0)),
        pl.BlockSpec((NB, k), lambda i: (i, 0)),
    ] + ([pl.BlockSpec((NB, HF), lambda i: (i, 0))] if has_last else []) + [
        pl.BlockSpec((HF, H), lambda i: (0, 0)),
        pl.BlockSpec((H, HF), lambda i: (0, 0)),
        pl.BlockSpec((1, HF), lambda i: (0, 0)),
        pl.BlockSpec((1, HF), lambda i: (0, 0)),
        pl.BlockSpec((HF, HF), lambda i: (0, 0)),
        pl.BlockSpec((HF,), lambda i: (0,)),
        pl.BlockSpec((k, HF), lambda i: (0, 0)),
        pl.BlockSpec((HF,), lambda i: (0,)),
    ]
    body = functools.partial(_c1_body, has_last) if has_last else _c1_nolast
    return pl.pallas_call(
        body,
        out_shape=[
            jax.ShapeDtypeStruct((NP, HF), _f32),
            jax.ShapeDtypeStruct((G, 1, HF), _f32),
            jax.ShapeDtypeStruct((G, 1, HF), _f32),
        ],
        grid=(G,),
        in_specs=in_specs,
        out_specs=[
            pl.BlockSpec((NB, HF), lambda i: (i, 0)),
            pl.BlockSpec((1, 1, HF), lambda i: (i, 0, 0)),
            pl.BlockSpec((1, 1, HF), lambda i: (i, 0, 0)),
        ],
    )(*args)


def _c1_nolast(agg_ref, h_ref, one8_ref, exp8_ref, sc_ref, of_ref,
               wagg_ref, bagg_ref, wdst_ref, bdst_ref, hf_ref, ps_ref, pq_ref):
    _c1_body(False, agg_ref, h_ref, None, one8_ref, exp8_ref, sc_ref, of_ref,
             wagg_ref, bagg_ref, wdst_ref, bdst_ref, hf_ref, ps_ref, pq_ref)


def _c2_body(hf_ref, ps_ref, pq_ref, g_ref, b_ref, o_ref):
    mu = jnp.sum(ps_ref[...][:, 0, :], axis=0) * (1.0 / N)
    ex2 = jnp.sum(pq_ref[...][:, 0, :], axis=0) * (1.0 / N)
    var = ex2 - mu * mu
    s = g_ref[...][0] / jnp.sqrt(var + 1e-5)
    o_ref[...] = jnp.maximum((hf_ref[...] - mu) * s + b_ref[...][0], 0.0)


def _c2(hf, ps, pq, g, b):
    return pl.pallas_call(
        _c2_body,
        out_shape=jax.ShapeDtypeStruct((NP, HF), _f32),
        grid=(G,),
        in_specs=[
            pl.BlockSpec((NB, HF), lambda i: (i, 0)),
            pl.BlockSpec((G, 1, HF), lambda i: (0, 0, 0)),
            pl.BlockSpec((G, 1, HF), lambda i: (0, 0, 0)),
            pl.BlockSpec((1, HF), lambda i: (0, 0)),
            pl.BlockSpec((1, HF), lambda i: (0, 0)),
        ],
        out_specs=pl.BlockSpec((NB, HF), lambda i: (i, 0)),
    )(hf, ps, pq, g, b)


def _pred_body(h_ref, w_ref, b_ref, o_ref):
    o_ref[...] = jnp.dot(h_ref[...], w_ref[...], preferred_element_type=_f32) + b_ref[...]


def _pred(h, wt, bp):
    return pl.pallas_call(
        _pred_body,
        out_shape=jax.ShapeDtypeStruct((NP, 128), _f32),
        grid=(G,),
        in_specs=[
            pl.BlockSpec((NB, HF), lambda i: (i, 0)),
            pl.BlockSpec((HF, 128), lambda i: (0, 0)),
            pl.BlockSpec((128,), lambda i: (0,)),
        ],
        out_specs=pl.BlockSpec((NB, 128), lambda i: (i, 0)),
    )(h, wt, bp)


def _sc_agg_body(fsrc_hbm, adst_hbm, pay_hbm, src_hbm, starts_hbm, out_hbm,
                 acc, rows0, rows1, pay0, pay1, si0, si1, adst_rng,
                 coeff_b, starts_v,
                 sem_r0, sem_r1, sem_s0, sem_s1, sem_p0, sem_p1):
    rows = (rows0, rows1)
    payb = (pay0, pay1)
    si = (si0, si1)
    sem_r = (sem_r0, sem_r1)
    sem_s = (sem_s0, sem_s1)
    sem_p = (sem_p0, sem_p1)
    wid = lax.axis_index("s") * 2 + lax.axis_index("c")
    pltpu.sync_copy(starts_hbm, starts_v)
    zero16 = jnp.zeros((16,), _f32)
    iota = lax.iota(_i32, 16)
    low = iota < 8
    col8 = lax.bitwise_and(iota, 7)
    zeros16i = jnp.zeros((16,), _i32)

    for p in range(2):
        r = wid * 2 + p

        def zrow(i, _):
            for vv in range(HF // 16):
                acc[i, pl.ds(vv * 16, 16)] = zero16
            return 0
        lax.fori_loop(0, RN, zrow, 0, unroll=False)

        pltpu.sync_copy(adst_hbm.at[pl.ds(pl.multiple_of(r * RN, 8), RN)],
                        adst_rng)
        sv = starts_v[pl.ds(r, 16)]
        es = sv[0]
        ee = sv[1]
        base0 = es - lax.rem(es, 8)
        nchunks = lax.div(ee - base0 + (C - 1), C)
        m = lax.div(nchunks + 1, 2)

        def cslice(ref, q):
            return ref.at[pl.ds(pl.multiple_of(base0 + q * C, 8), C)]

        pltpu.async_copy(cslice(src_hbm, 0), si[0], sem_s[0]).wait()
        pltpu.async_copy(fsrc_hbm.at[si[0]], rows[0], sem_r[0])
        pltpu.async_copy(cslice(pay_hbm, 0), payb[0], sem_p[0])
        pltpu.async_copy(cslice(src_hbm, 1), si[1], sem_s[1])

        def pair_iter(i, _):
            for b in range(2):
                k = 2 * i + b
                nb = 1 - b
                pltpu.make_async_copy(cslice(src_hbm, k + 1), si[nb],
                                      sem_s[nb]).wait()
                pltpu.async_copy(fsrc_hbm.at[si[nb]], rows[nb], sem_r[nb])
                pltpu.make_async_copy(cslice(pay_hbm, k), payb[b],
                                      sem_p[b]).wait()
                pltpu.make_async_copy(fsrc_hbm.at[si[b]], rows[b],
                                      sem_r[b]).wait()
                pltpu.async_copy(cslice(src_hbm, k + 2), si[b], sem_s[b])
                pltpu.async_copy(cslice(pay_hbm, k + 1), payb[nb], sem_p[nb])
                cbase = base0 + k * C
                jlo = jnp.maximum(es - cbase, 0)
                jhi = jnp.minimum(ee - cbase, C)
                rw = rows[b]
                pw = payb[b]

                def pair_body(pp, _):
                    rlo = jnp.where(low, 2 * pp, 2 * pp + 1)
                    a = plsc.load_gather(rw, [rlo, col8 + HF])
                    ev = plsc.load_gather(pw, [rlo, col8 + 1])
                    drow = plsc.bitcast(
                        plsc.load_gather(pw, [rlo, zeros16i]), _i32)
                    dv = plsc.load_gather(adst_rng, [drow, col8])
                    sm = a + dv + ev
                    coeff_b[pl.ds(pp * 16, 16)] = jnp.maximum(sm, 0.2 * sm)
                    return 0

                lax.fori_loop(lax.div(jlo, 2), lax.div(jhi + 1, 2),
                              pair_body, 0, unroll=False)

                def edge_body(j, _):
                    dl = plsc.bitcast(pw[j, pl.ds(0, 16)], _i32)[0]
                    cvec = coeff_b[pl.ds(j * 8, 16)]
                    for hh in range(H):
                        cv = lax.broadcast(cvec[hh], (16,))
                        for vv in range(4):
                            off = hh * 64 + vv * 16
                            plsc.addupdate(acc.at[dl, pl.ds(off, 16)],
                                           cv * rw[j, pl.ds(off, 16)])
                    return 0

                lax.fori_loop(jlo, jhi, edge_body, 0, unroll=False)
            return 0

        lax.fori_loop(0, m, pair_iter, 0, unroll=False)
        pltpu.make_async_copy(fsrc_hbm.at[si[0]], rows[0], sem_r[0]).wait()
        pltpu.make_async_copy(cslice(src_hbm, 0), si[1], sem_s[1]).wait()
        pltpu.make_async_copy(cslice(pay_hbm, 0), payb[0], sem_p[0]).wait()
        pltpu.sync_copy(acc, out_hbm.at[r])


def _sc_agg(fsrc, adst, pay, src_p, starts_p):
    return pl.kernel(
        _sc_agg_body,
        out_type=jax.ShapeDtypeStruct((NRANGES, RN, HF), _f32),
        mesh=plsc.VectorSubcoreMesh(core_axis_name="c", subcore_axis_name="s",
                                    num_cores=2, num_subcores=16),
        compiler_params=pltpu.CompilerParams(needs_layout_passes=False),
        scratch_types=[
            pltpu.VMEM((RN, HF), _f32),
            pltpu.VMEM((C, FEXT), _f32),
            pltpu.VMEM((C, FEXT), _f32),
            pltpu.VMEM((C, 16), _f32),
            pltpu.VMEM((C, 16), _f32),
            pltpu.VMEM((C,), _i32),
            pltpu.VMEM((C,), _i32),
            pltpu.VMEM((RN, H), _f32),
            pltpu.VMEM((C * H + 16,), _f32),
            pltpu.VMEM((80,), _i32),
            pltpu.SemaphoreType.DMA,
            pltpu.SemaphoreType.DMA,
            pltpu.SemaphoreType.DMA,
            pltpu.SemaphoreType.DMA,
            pltpu.SemaphoreType.DMA,
            pltpu.SemaphoreType.DMA,
        ],
    )(fsrc, adst, pay, src_p, starts_p)


def _padw(w, k):
    wt = w.T
    if wt.shape[0] < k:
        wt = jnp.pad(wt, ((0, k - wt.shape[0]), (0, 0)))
    return wt


def kernel(x, edge_index, edge_attr, params):
    src = edge_index[0]
    dst = edge_index[1]

    order = jnp.argsort(dst)
    dst_s = dst[order]
    src_p = jnp.pad(src[order], (0, EP - E))
    dstloc_f = lax.bitcast_convert_type(
        jnp.pad(dst_s % RN, (0, EP - E)), _f32).reshape(EP, 1)
    ea_p = jnp.pad(edge_attr[order], ((0, EP - E), (0, 0)))
    starts = jnp.searchsorted(dst_s, jnp.arange(65, dtype=_i32) * RN).astype(_i32)
    starts_p = jnp.pad(starts, (0, 15))

    xp = jnp.pad(x, ((0, NP - N), (0, 0)))
    p = params

    one8 = (jnp.arange(HF, dtype=_i32)[:, None] // F
            == jnp.arange(H, dtype=_i32)[None, :]).astype(_f32)
    exp8 = one8.T

    enc_wt = jnp.pad(p['node_enc_w'].T, ((0, 0), (0, K0 - FIRST)))
    enc_b = jnp.pad(p['node_enc_b'], (0, K0 - FIRST))
    h = _node_enc(xp, enc_wt, enc_b)

    h_last = None
    for i in range(2):
        k = K0 if i == 0 else HF
        wf = _padw(p[f'src_fc_w_{i}'], k)
        was = _padw(p[f'attn_src_w_{i}'], k)
        wad = _padw(p[f'attn_dst_w_{i}'], k)
        wfe = jnp.concatenate([wf, was, jnp.zeros((k, FEXT - HF - H), _f32)],
                              axis=1)
        fsrc, adst = _proj(h, wfe, wad)

        w1t = p[f'edge_enc_w_{i}'].T
        b1 = p[f'edge_enc_b_{i}']
        w2t = p[f'attn_edge_w_{i}'].T
        pay = _eatt(ea_p, dstloc_f, w1t, b1, w2t)

        agg = _sc_agg(fsrc, adst, pay, src_p, starts_p).reshape(NP, HF)

        scale = p[f'scale_{i}'].reshape(1, HF)
        offset = p[f'offset_{i}'].reshape(1, HF)
        wagg = p[f'agg_w_{i}'].T
        bagg = p[f'agg_b_{i}']
        wdst = _padw(p[f'dst_w_{i}'], k)
        bdst = p[f'dst_b_{i}']
        hf, ps, pq = _c1(agg, h, h_last, one8, exp8, scale, offset,
                         wagg, bagg, wdst, bdst)
        h = _c2(hf, ps, pq, p[f'bn_g_{i}'].reshape(1, HF),
                p[f'bn_b_{i}'].reshape(1, HF))
        h_last = hf

    pred_wt = jnp.pad(p['pred_w'].T, ((0, 0), (0, 128 - NC_OUT)))
    pred_b = jnp.pad(p['pred_b'], (0, 128 - NC_OUT))
    out = _pred(h, pred_wt, pred_b)
    return out[:N, :NC_OUT]


# speedup vs baseline: 14.9729x; 1.0237x over previous
import functools

import jax
import jax.numpy as jnp
from jax import lax
from jax.experimental import pallas as pl
from jax.experimental.pallas import tpu as pltpu
from jax.experimental.pallas import tpu_sc as plsc

N = 10000
NP = 10240
NRANGES = 64
RN = 160
E = 160000
EP = 160768
C = 16
H = 8
F = 64
HF = 512
D_IN = 256
FIRST = 150
K0 = 256
NB = 1024
G = NP // NB
NC_OUT = 40
FEXT = 640

_f32 = jnp.float32
_i32 = jnp.int32


def _enc_body(x_ref, w_ref, b_ref, o_ref):
    o_ref[...] = jax.nn.relu(
        jnp.dot(x_ref[...], w_ref[...], preferred_element_type=_f32) + b_ref[...]
    )


def _node_enc(xp, wt, bp):
    return pl.pallas_call(
        _enc_body,
        out_shape=jax.ShapeDtypeStruct((NP, K0), _f32),
        grid=(G,),
        in_specs=[
            pl.BlockSpec((NB, D_IN), lambda i: (i, 0)),
            pl.BlockSpec((D_IN, K0), lambda i: (0, 0)),
            pl.BlockSpec((K0,), lambda i: (0,)),
        ],
        out_specs=pl.BlockSpec((NB, K0), lambda i: (i, 0)),
    )(xp, wt, bp)


def _proj_body(h_ref, wfe_ref, wad_ref, f_ref, ad_ref):
    hv = h_ref[...]
    f_ref[...] = jnp.dot(hv, wfe_ref[...], preferred_element_type=_f32)
    ad_ref[...] = jnp.dot(hv, wad_ref[...], preferred_element_type=_f32)


def _proj(h, wfe, wad):
    k = h.shape[1]
    return pl.pallas_call(
        _proj_body,
        out_shape=[
            jax.ShapeDtypeStruct((NP, FEXT), _f32),
            jax.ShapeDtypeStruct((NP, H), _f32),
        ],
        grid=(G,),
        in_specs=[
            pl.BlockSpec((NB, k), lambda i: (i, 0)),
            pl.BlockSpec((k, FEXT), lambda i: (0, 0)),
            pl.BlockSpec((k, H), lambda i: (0, 0)),
        ],
        out_specs=[
            pl.BlockSpec((NB, FEXT), lambda i: (i, 0)),
            pl.BlockSpec((NB, H), lambda i: (i, 0)),
        ],
    )(h, wfe, wad)


def _eatt_body(ea_ref, dl_ref, w1_ref, b1_ref, w2_ref, o_ref):
    t = jax.nn.relu(
        jnp.dot(ea_ref[...], w1_ref[...], preferred_element_type=_f32) + b1_ref[...]
    )
    ea = jnp.dot(t, w2_ref[...], preferred_element_type=_f32)
    o_ref[...] = jnp.concatenate(
        [dl_ref[...], ea, jnp.zeros((NB, 7), _f32)], axis=1)


def _eatt(ea_p, dl_f, w1t, b1, w2t):
    return pl.pallas_call(
        _eatt_body,
        out_shape=jax.ShapeDtypeStruct((EP, 16), _f32),
        grid=(EP // NB,),
        in_specs=[
            pl.BlockSpec((NB, 16), lambda i: (i, 0)),
            pl.BlockSpec((NB, 1), lambda i: (i, 0)),
            pl.BlockSpec((16, 16), lambda i: (0, 0)),
            pl.BlockSpec((16,), lambda i: (0,)),
            pl.BlockSpec((16, H), lambda i: (0, 0)),
        ],
        out_specs=pl.BlockSpec((NB, 16), lambda i: (i, 0)),
    )(ea_p, dl_f, w1t, b1, w2t)


def _c1_body(has_last, agg_ref, h_ref, hl_ref, one8_ref, exp8_ref, sc_ref, of_ref,
             wagg_ref, bagg_ref, wdst_ref, bdst_ref, hf_ref, ps_ref, pq_ref):
    a = agg_ref[...]
    m8 = jnp.dot(a, one8_ref[...], preferred_element_type=_f32) * (1.0 / F)
    q8 = jnp.dot(a * a, one8_ref[...], preferred_element_type=_f32) * (1.0 / F)
    mean = jnp.dot(m8, exp8_ref[...], preferred_element_type=_f32)
    msq = jnp.dot(q8, exp8_ref[...], preferred_element_type=_f32)
    var = msq - mean * mean + 1e-9
    hn = (a - mean) * sc_ref[...] * lax.rsqrt(var) + of_ref[...]
    o = (jnp.dot(hn, wagg_ref[...], preferred_element_type=_f32) + bagg_ref[...]
         + jnp.dot(h_ref[...], wdst_ref[...], preferred_element_type=_f32)
         + bdst_ref[...])
    if has_last:
        o = o + hl_ref[...]
    hf_ref[...] = o
    gi = pl.program_id(0)
    rows = lax.broadcasted_iota(_i32, (NB, 1), 0) + gi * NB
    om = jnp.where(rows < N, o, 0.0)
    ps_ref[...] = jnp.sum(om, axis=0, keepdims=True).reshape(1, 1, HF)
    pq_ref[...] = jnp.sum(om * om, axis=0, keepdims=True).reshape(1, 1, HF)


def _c1(agg, h, hl, one8, exp8, scale, offset, wagg, bagg, wdst, bdst):
    k = h.shape[1]
    has_last = hl is not None
    args = [agg, h] + ([hl] if has_last else []) + [
        one8, exp8, scale, offset, wagg, bagg, wdst, bdst]
    in_specs = [
        pl.BlockSpec((NB, HF), lambda i: (i, 0)),
        pl.BlockSpec((NB, k), lambda i: (i, 0)),
    ] + ([pl.BlockSpec((NB, HF), lambda i: (i, 0))] if has_last else []) + [
        pl.BlockSpec((HF, H), lambda i: (0, 0)),
        pl.BlockSpec((H, HF), lambda i: (0, 0)),
        pl.BlockSpec((1, HF), lambda i: (0, 0)),
        pl.BlockSpec((1, HF), lambda i: (0, 0)),
        pl.BlockSpec((HF, HF), lambda i: (0, 0)),
        pl.BlockSpec((HF,), lambda i: (0,)),
        pl.BlockSpec((k, HF), lambda i: (0, 0)),
        pl.BlockSpec((HF,), lambda i: (0,)),
    ]
    body = functools.partial(_c1_body, has_last) if has_last else _c1_nolast
    return pl.pallas_call(
        body,
        out_shape=[
            jax.ShapeDtypeStruct((NP, HF), _f32),
            jax.ShapeDtypeStruct((G, 1, HF), _f32),
            jax.ShapeDtypeStruct((G, 1, HF), _f32),
        ],
        grid=(G,),
        in_specs=in_specs,
        out_specs=[
            pl.BlockSpec((NB, HF), lambda i: (i, 0)),
            pl.BlockSpec((1, 1, HF), lambda i: (i, 0, 0)),
            pl.BlockSpec((1, 1, HF), lambda i: (i, 0, 0)),
        ],
    )(*args)


def _c1_nolast(agg_ref, h_ref, one8_ref, exp8_ref, sc_ref, of_ref,
               wagg_ref, bagg_ref, wdst_ref, bdst_ref, hf_ref, ps_ref, pq_ref):
    _c1_body(False, agg_ref, h_ref, None, one8_ref, exp8_ref, sc_ref, of_ref,
             wagg_ref, bagg_ref, wdst_ref, bdst_ref, hf_ref, ps_ref, pq_ref)


def _c2_body(hf_ref, ps_ref, pq_ref, g_ref, b_ref, o_ref):
    mu = jnp.sum(ps_ref[...][:, 0, :], axis=0) * (1.0 / N)
    ex2 = jnp.sum(pq_ref[...][:, 0, :], axis=0) * (1.0 / N)
    var = ex2 - mu * mu
    s = g_ref[...][0] / jnp.sqrt(var + 1e-5)
    o_ref[...] = jnp.maximum((hf_ref[...] - mu) * s + b_ref[...][0], 0.0)


def _c2(hf, ps, pq, g, b):
    return pl.pallas_call(
        _c2_body,
        out_shape=jax.ShapeDtypeStruct((NP, HF), _f32),
        grid=(G,),
        in_specs=[
            pl.BlockSpec((NB, HF), lambda i: (i, 0)),
            pl.BlockSpec((G, 1, HF), lambda i: (0, 0, 0)),
            pl.BlockSpec((G, 1, HF), lambda i: (0, 0, 0)),
            pl.BlockSpec((1, HF), lambda i: (0, 0)),
            pl.BlockSpec((1, HF), lambda i: (0, 0)),
        ],
        out_specs=pl.BlockSpec((NB, HF), lambda i: (i, 0)),
    )(hf, ps, pq, g, b)


def _pred_body(h_ref, w_ref, b_ref, o_ref):
    o_ref[...] = jnp.dot(h_ref[...], w_ref[...], preferred_element_type=_f32) + b_ref[...]


def _pred(h, wt, bp):
    return pl.pallas_call(
        _pred_body,
        out_shape=jax.ShapeDtypeStruct((NP, 128), _f32),
        grid=(G,),
        in_specs=[
            pl.BlockSpec((NB, HF), lambda i: (i, 0)),
            pl.BlockSpec((HF, 128), lambda i: (0, 0)),
            pl.BlockSpec((128,), lambda i: (0,)),
        ],
        out_specs=pl.BlockSpec((NB, 128), lambda i: (i, 0)),
    )(h, wt, bp)


def _sc_agg_body(fsrc_hbm, adst_hbm, pay_hbm, src_hbm, starts_hbm, out_hbm,
                 acc, rows0, rows1, pay0, pay1, si0, si1, adst_rng,
                 coeff_b, starts_v,
                 sem_r0, sem_r1, sem_s0, sem_s1, sem_p0, sem_p1):
    rows = (rows0, rows1)
    payb = (pay0, pay1)
    si = (si0, si1)
    sem_r = (sem_r0, sem_r1)
    sem_s = (sem_s0, sem_s1)
    sem_p = (sem_p0, sem_p1)
    wid = lax.axis_index("s") * 2 + lax.axis_index("c")
    pltpu.sync_copy(starts_hbm, starts_v)
    zero16 = jnp.zeros((16,), _f32)
    iota = lax.iota(_i32, 16)
    low = iota < 8
    col8 = lax.bitwise_and(iota, 7)
    zeros16i = jnp.zeros((16,), _i32)

    for p in range(2):
        r = wid * 2 + p

        def zrow(i, _):
            for vv in range(HF // 16):
                acc[i, pl.ds(vv * 16, 16)] = zero16
            return 0
        lax.fori_loop(0, RN, zrow, 0, unroll=False)

        pltpu.sync_copy(adst_hbm.at[pl.ds(pl.multiple_of(r * RN, 8), RN)],
                        adst_rng)
        sv = starts_v[pl.ds(r, 16)]
        es = sv[0]
        ee = sv[1]
        base0 = es - lax.rem(es, 8)
        nchunks = lax.div(ee - base0 + (C - 1), C)
        m = lax.div(nchunks + 1, 2)

        def cslice(ref, q):
            return ref.at[pl.ds(pl.multiple_of(base0 + q * C, 8), C)]

        pltpu.async_copy(cslice(src_hbm, 0), si[0], sem_s[0]).wait()
        pltpu.async_copy(fsrc_hbm.at[si[0]], rows[0], sem_r[0])
        pltpu.async_copy(cslice(pay_hbm, 0), payb[0], sem_p[0])
        pltpu.async_copy(cslice(src_hbm, 1), si[1], sem_s[1])

        def pair_iter(i, _):
            for b in range(2):
                k = 2 * i + b
                nb = 1 - b
                pltpu.make_async_copy(cslice(src_hbm, k + 1), si[nb],
                                      sem_s[nb]).wait()
                pltpu.async_copy(fsrc_hbm.at[si[nb]], rows[nb], sem_r[nb])
                pltpu.make_async_copy(cslice(pay_hbm, k), payb[b],
                                      sem_p[b]).wait()
                pltpu.make_async_copy(fsrc_hbm.at[si[b]], rows[b],
                                      sem_r[b]).wait()
                pltpu.async_copy(cslice(src_hbm, k + 2), si[b], sem_s[b])
                pltpu.async_copy(cslice(pay_hbm, k + 1), payb[nb], sem_p[nb])
                cbase = base0 + k * C
                jlo = jnp.maximum(es - cbase, 0)
                jhi = jnp.minimum(ee - cbase, C)
                rw = rows[b]
                pw = payb[b]

                def pair_body(pp, _):
                    rlo = jnp.where(low, 2 * pp, 2 * pp + 1)
                    a = plsc.load_gather(rw, [rlo, col8 + HF])
                    ev = plsc.load_gather(pw, [rlo, col8 + 1])
                    drow = plsc.bitcast(
                        plsc.load_gather(pw, [rlo, zeros16i]), _i32)
                    dv = plsc.load_gather(adst_rng, [drow, col8])
                    sm = a + dv + ev
                    valid = jnp.logical_and(rlo >= jlo, rlo < jhi)
                    coeff_b[pl.ds(pp * 16, 16)] = jnp.where(
                        valid, jnp.maximum(sm, 0.2 * sm), 0.0)
                    return 0

                plo = lax.div(jlo, 2)
                phi = lax.div(jhi + 1, 2)
                lax.fori_loop(plo, phi, pair_body, 0, unroll=False)

                def epair_body(pp, _):
                    j0 = 2 * pp
                    j1 = j0 + 1
                    dl0 = plsc.bitcast(pw[j0, pl.ds(0, 16)], _i32)[0]
                    dl1 = plsc.bitcast(pw[j1, pl.ds(0, 16)], _i32)[0]
                    cpair = coeff_b[pl.ds(j0 * 8, 16)]
                    for hh in range(H):
                        c0 = lax.broadcast(cpair[hh], (16,))
                        c1 = lax.broadcast(cpair[hh + 8], (16,))
                        for vv in range(4):
                            off = hh * 64 + vv * 16
                            plsc.addupdate(acc.at[dl0, pl.ds(off, 16)],
                                           c0 * rw[j0, pl.ds(off, 16)])
                            plsc.addupdate(acc.at[dl1, pl.ds(off, 16)],
                                           c1 * rw[j1, pl.ds(off, 16)])
                    return 0

                lax.fori_loop(plo, phi, epair_body, 0, unroll=False)
            return 0

        lax.fori_loop(0, m, pair_iter, 0, unroll=False)
        pltpu.make_async_copy(fsrc_hbm.at[si[0]], rows[0], sem_r[0]).wait()
        pltpu.make_async_copy(cslice(src_hbm, 0), si[1], sem_s[1]).wait()
        pltpu.make_async_copy(cslice(pay_hbm, 0), payb[0], sem_p[0]).wait()
        pltpu.sync_copy(acc, out_hbm.at[r])


def _sc_agg(fsrc, adst, pay, src_p, starts_p):
    return pl.kernel(
        _sc_agg_body,
        out_type=jax.ShapeDtypeStruct((NRANGES, RN, HF), _f32),
        mesh=plsc.VectorSubcoreMesh(core_axis_name="c", subcore_axis_name="s",
                                    num_cores=2, num_subcores=16),
        compiler_params=pltpu.CompilerParams(needs_layout_passes=False),
        scratch_types=[
            pltpu.VMEM((RN, HF), _f32),
            pltpu.VMEM((C, FEXT), _f32),
            pltpu.VMEM((C, FEXT), _f32),
            pltpu.VMEM((C, 16), _f32),
            pltpu.VMEM((C, 16), _f32),
            pltpu.VMEM((C,), _i32),
            pltpu.VMEM((C,), _i32),
            pltpu.VMEM((RN, H), _f32),
            pltpu.VMEM((C * H + 16,), _f32),
            pltpu.VMEM((80,), _i32),
            pltpu.SemaphoreType.DMA,
            pltpu.SemaphoreType.DMA,
            pltpu.SemaphoreType.DMA,
            pltpu.SemaphoreType.DMA,
            pltpu.SemaphoreType.DMA,
            pltpu.SemaphoreType.DMA,
        ],
    )(fsrc, adst, pay, src_p, starts_p)


def _padw(w, k):
    wt = w.T
    if wt.shape[0] < k:
        wt = jnp.pad(wt, ((0, k - wt.shape[0]), (0, 0)))
    return wt


def kernel(x, edge_index, edge_attr, params):
    src = edge_index[0]
    dst = edge_index[1]

    order = jnp.argsort(dst)
    dst_s = dst[order]
    src_p = jnp.pad(src[order], (0, EP - E))
    dstloc_f = lax.bitcast_convert_type(
        jnp.pad(dst_s % RN, (0, EP - E)), _f32).reshape(EP, 1)
    ea_p = jnp.pad(edge_attr[order], ((0, EP - E), (0, 0)))
    starts = jnp.searchsorted(dst_s, jnp.arange(65, dtype=_i32) * RN).astype(_i32)
    starts_p = jnp.pad(starts, (0, 15))

    xp = jnp.pad(x, ((0, NP - N), (0, 0)))
    p = params

    one8 = (jnp.arange(HF, dtype=_i32)[:, None] // F
            == jnp.arange(H, dtype=_i32)[None, :]).astype(_f32)
    exp8 = one8.T

    enc_wt = jnp.pad(p['node_enc_w'].T, ((0, 0), (0, K0 - FIRST)))
    enc_b = jnp.pad(p['node_enc_b'], (0, K0 - FIRST))
    h = _node_enc(xp, enc_wt, enc_b)

    h_last = None
    for i in range(2):
        k = K0 if i == 0 else HF
        wf = _padw(p[f'src_fc_w_{i}'], k)
        was = _padw(p[f'attn_src_w_{i}'], k)
        wad = _padw(p[f'attn_dst_w_{i}'], k)
        wfe = jnp.concatenate([wf, was, jnp.zeros((k, FEXT - HF - H), _f32)],
                              axis=1)
        fsrc, adst = _proj(h, wfe, wad)

        w1t = p[f'edge_enc_w_{i}'].T
        b1 = p[f'edge_enc_b_{i}']
        w2t = p[f'attn_edge_w_{i}'].T
        pay = _eatt(ea_p, dstloc_f, w1t, b1, w2t)

        agg = _sc_agg(fsrc, adst, pay, src_p, starts_p).reshape(NP, HF)

        scale = p[f'scale_{i}'].reshape(1, HF)
        offset = p[f'offset_{i}'].reshape(1, HF)
        wagg = p[f'agg_w_{i}'].T
        bagg = p[f'agg_b_{i}']
        wdst = _padw(p[f'dst_w_{i}'], k)
        bdst = p[f'dst_b_{i}']
        hf, ps, pq = _c1(agg, h, h_last, one8, exp8, scale, offset,
                         wagg, bagg, wdst, bdst)
        h = _c2(hf, ps, pq, p[f'bn_g_{i}'].reshape(1, HF),
                p[f'bn_b_{i}'].reshape(1, HF))
        h_last = hf

    pred_wt = jnp.pad(p['pred_w'].T, ((0, 0), (0, 128 - NC_OUT)))
    pred_b = jnp.pad(p['pred_b'], (0, 128 - NC_OUT))
    out = _pred(h, pred_wt, pred_b)
    return out[:N, :NC_OUT]
